# baseline (speedup 1.0000x reference)
import numpy as np

import jax
import jax.numpy as jnp
from jax.experimental import pallas as pl

_B = 2048
_BBLK = 512
_FLAT = 3072
_NSEL = 1024
_NENS = 160
_NCOMB = _NSEL + _NENS
_K = 4
_OUT_DIM = 16
_NCLS = 10


def _build_a1p(W1):
    k = np.arange(4)
    i = np.arange(14)
    j = np.arange(14)
    c = np.arange(3)
    u = np.arange(5)
    v = np.arange(5)
    K_, I_, J_, C_, U_, V_ = np.meshgrid(k, i, j, c, u, v, indexing="ij")
    dy, dx = K_ // 2, K_ % 2
    rows = (C_ * 1024 + (2 * I_ + dy + U_) * 32 + (2 * J_ + dx + V_)).ravel()
    cols = (K_ * 256 + I_ * 14 + J_).ravel()
    vals = W1[0, C_.ravel(), U_.ravel(), V_.ravel()]
    return jnp.zeros((_FLAT, _NSEL), jnp.float32).at[rows, cols].set(vals)


def _build_a2p(W2):
    k = np.arange(4)
    m = np.arange(2)
    i = np.arange(5)
    j = np.arange(5)
    u = np.arange(5)
    v = np.arange(5)
    K_, M_, I_, J_, U_, V_ = np.meshgrid(k, m, i, j, u, v, indexing="ij")
    dy, dx = K_ // 2, K_ % 2
    rows = ((2 * I_ + dy + U_) * 14 + (2 * J_ + dx + V_)).ravel()
    cols = (K_ * 128 + M_ * 25 + I_ * 5 + J_).ravel()
    vals = W2[M_.ravel(), 0, U_.ravel(), V_.ravel()]
    return jnp.zeros((256, 512), jnp.float32).at[rows, cols].set(vals)


def _build_a3p(W3):
    n = np.arange(4)
    m = np.arange(2)
    u = np.arange(5)
    v = np.arange(5)
    N_, M_, U_, V_ = np.meshgrid(n, m, u, v, indexing="ij")
    rows = (M_ * 25 + U_ * 5 + V_).ravel()
    cols = N_.ravel()
    vals = W3[N_.ravel(), M_.ravel(), U_.ravel(), V_.ravel()]
    return jnp.zeros((128, 4), jnp.float32).at[rows, cols].set(vals)


_E_EXPAND = np.kron(np.eye(_OUT_DIM, dtype=np.float32), np.ones((1, _NCLS), np.float32))
_S_SUM = np.kron(np.ones((_OUT_DIM, 1), np.float32), np.eye(_NCLS, dtype=np.float32))


def _fwd_kernel(x_ref, acomb_ref, a2p_ref, a3p_ref, wl_ref, b1_ref, b2v_ref,
                b3_ref, bl_ref, e_ref, s_ref, blearn_ref, out_ref):
    xb = x_ref[...]
    y = jnp.dot(xb, acomb_ref[...], preferred_element_type=jnp.float32)
    y1 = y[:, :_NSEL]
    g = y[:, _NSEL:]
    p1 = jnp.maximum(jnp.maximum(y1[:, 0:256], y1[:, 256:512]),
                     jnp.maximum(y1[:, 512:768], y1[:, 768:1024]))
    p1 = jax.nn.relu(p1 + b1_ref[0, 0])
    y2 = jnp.dot(p1, a2p_ref[...], preferred_element_type=jnp.float32)
    p2 = jnp.maximum(jnp.maximum(y2[:, 0:128], y2[:, 128:256]),
                     jnp.maximum(y2[:, 256:384], y2[:, 384:512]))
    p2 = jax.nn.relu(p2 + b2v_ref[...])
    h4 = jax.nn.relu(jnp.dot(p2, a3p_ref[...], preferred_element_type=jnp.float32)
                     + b3_ref[...])
    s = jnp.dot(h4, wl_ref[...], preferred_element_type=jnp.float32) + bl_ref[...]
    sp = s[:, :, None]
    sl = s[:, None, :]
    ip = jax.lax.broadcasted_iota(jnp.int32, (s.shape[0], 16, 16), 1)
    il = jax.lax.broadcasted_iota(jnp.int32, (s.shape[0], 16, 16), 2)
    pred = (sp > sl) | ((sp == sl) & (ip < il))
    cnt = pred.astype(jnp.float32).sum(axis=1)
    mask = (cnt < (_K - 0.5)).astype(jnp.float32)
    maske = jnp.dot(mask, e_ref[...], preferred_element_type=jnp.float32)
    out = jnp.dot(g * maske, s_ref[...], preferred_element_type=jnp.float32)
    out = out + jnp.dot(mask, blearn_ref[...], preferred_element_type=jnp.float32)
    out_ref[...] = out


def kernel(x, W1, b1, W2, b2, W3, b3, Wl, bl, Wlearn, blearn):
    B = x.shape[0]
    xflat = x.reshape(B, _FLAT)
    a1p = _build_a1p(W1)
    wall = Wlearn.transpose(1, 0, 2).reshape(_FLAT, _NENS)
    acomb = jnp.concatenate([a1p, wall], axis=1)
    a2p = _build_a2p(W2)
    a3p = _build_a3p(W3)
    m = np.arange(2)
    q = np.arange(25)
    M_, Q_ = np.meshgrid(m, q, indexing="ij")
    b2v = jnp.zeros((1, 128), jnp.float32).at[0, (M_ * 25 + Q_).ravel()].set(b2[M_.ravel()])
    b1r = b1.reshape(1, 1)
    b3r = b3.reshape(1, 4)
    blr = bl.reshape(1, _OUT_DIM)
    e_m = jnp.asarray(_E_EXPAND)
    s_m = jnp.asarray(_S_SUM)

    out = pl.pallas_call(
        _fwd_kernel,
        grid=(B // _BBLK,),
        in_specs=[
            pl.BlockSpec((_BBLK, _FLAT), lambda i: (i, 0)),
            pl.BlockSpec((_FLAT, _NCOMB), lambda i: (0, 0)),
            pl.BlockSpec((256, 512), lambda i: (0, 0)),
            pl.BlockSpec((128, 4), lambda i: (0, 0)),
            pl.BlockSpec((4, _OUT_DIM), lambda i: (0, 0)),
            pl.BlockSpec((1, 1), lambda i: (0, 0)),
            pl.BlockSpec((1, 128), lambda i: (0, 0)),
            pl.BlockSpec((1, 4), lambda i: (0, 0)),
            pl.BlockSpec((1, _OUT_DIM), lambda i: (0, 0)),
            pl.BlockSpec((_OUT_DIM, _NENS), lambda i: (0, 0)),
            pl.BlockSpec((_NENS, _NCLS), lambda i: (0, 0)),
            pl.BlockSpec((_OUT_DIM, _NCLS), lambda i: (0, 0)),
        ],
        out_specs=pl.BlockSpec((_BBLK, _NCLS), lambda i: (i, 0)),
        out_shape=jax.ShapeDtypeStruct((B, _NCLS), jnp.float32),
    )(xflat, acomb, a2p, a3p, Wl, b1r, b2v, b3r, blr, e_m, s_m, blearn)
    return out


# speedup vs baseline: 4.8047x; 4.8047x over previous
import numpy as np

import jax
import jax.numpy as jnp
from jax.experimental import pallas as pl

_B = 2048
_BBLK = 512
_FLAT = 3072
_NSEL = 1024
_NENS = 160
_NCOMB = _NSEL + _NENS
_K = 4
_OUT_DIM = 16
_NCLS = 10


def _sel(h_in, n_out, strides=2):
    d = np.arange(2)[:, None, None, None]
    y = np.arange(h_in)[None, :, None, None]
    i = np.arange((h_in - 4) // strides if strides == 2 else n_out)[None, None, :, None]
    u = np.arange(5)[None, None, None, :]
    return (y == strides * i + d + u).astype(np.float32)


_RY1 = _sel(32, 14)
_RY2 = _sel(14, 5)


def _build_a1p(W1):
    w = W1[0]
    t1 = jnp.einsum("cuv,ayiu->cayiv", w, jnp.asarray(_RY1))
    a1 = jnp.einsum("cayiv,bxjv->cyxabij", t1, jnp.asarray(_RY1))
    a1 = a1.reshape(_FLAT, 4, 196)
    a1 = jnp.pad(a1, ((0, 0), (0, 0), (0, 60)))
    return a1.reshape(_FLAT, _NSEL)


def _build_a2p(W2):
    w = W2[:, 0]
    t1 = jnp.einsum("muv,ayiu->mayiv", w, jnp.asarray(_RY2))
    a2 = jnp.einsum("mayiv,bxjv->yxabmij", t1, jnp.asarray(_RY2))
    a2 = a2.reshape(196, 4, 50)
    a2 = jnp.pad(a2, ((0, 60), (0, 0), (0, 78)))
    return a2.reshape(256, 512)


def _build_a3p(W3):
    a3 = W3.transpose(1, 2, 3, 0).reshape(50, 4)
    return jnp.pad(a3, ((0, 78), (0, 0)))


_E_EXPAND = np.kron(np.eye(_OUT_DIM, dtype=np.float32), np.ones((1, _NCLS), np.float32))
_S_SUM = np.kron(np.ones((_OUT_DIM, 1), np.float32), np.eye(_NCLS, dtype=np.float32))


def _fwd_kernel(x_ref, acomb_ref, a2p_ref, a3p_ref, wl_ref, b1_ref, b2v_ref,
                b3_ref, bl_ref, e_ref, s_ref, blearn_ref, out_ref):
    xb = x_ref[...]
    y = jnp.dot(xb, acomb_ref[...], preferred_element_type=jnp.float32)
    y1 = y[:, :_NSEL]
    g = y[:, _NSEL:]
    p1 = jnp.maximum(jnp.maximum(y1[:, 0:256], y1[:, 256:512]),
                     jnp.maximum(y1[:, 512:768], y1[:, 768:1024]))
    p1 = jax.nn.relu(p1 + b1_ref[0, 0])
    y2 = jnp.dot(p1, a2p_ref[...], preferred_element_type=jnp.float32)
    p2 = jnp.maximum(jnp.maximum(y2[:, 0:128], y2[:, 128:256]),
                     jnp.maximum(y2[:, 256:384], y2[:, 384:512]))
    p2 = jax.nn.relu(p2 + b2v_ref[...])
    h4 = jax.nn.relu(jnp.dot(p2, a3p_ref[...], preferred_element_type=jnp.float32)
                     + b3_ref[...])
    s = jnp.dot(h4, wl_ref[...], preferred_element_type=jnp.float32) + bl_ref[...]
    sp = s[:, :, None]
    sl = s[:, None, :]
    ip = jax.lax.broadcasted_iota(jnp.int32, (s.shape[0], 16, 16), 1)
    il = jax.lax.broadcasted_iota(jnp.int32, (s.shape[0], 16, 16), 2)
    pred = (sp > sl) | ((sp == sl) & (ip < il))
    cnt = pred.astype(jnp.float32).sum(axis=1)
    mask = (cnt < (_K - 0.5)).astype(jnp.float32)
    maske = jnp.dot(mask, e_ref[...], preferred_element_type=jnp.float32)
    out = jnp.dot(g * maske, s_ref[...], preferred_element_type=jnp.float32)
    out = out + jnp.dot(mask, blearn_ref[...], preferred_element_type=jnp.float32)
    out_ref[...] = out


def kernel(x, W1, b1, W2, b2, W3, b3, Wl, bl, Wlearn, blearn):
    B = x.shape[0]
    xflat = x.reshape(B, _FLAT)
    a1p = _build_a1p(W1)
    wall = Wlearn.transpose(1, 0, 2).reshape(_FLAT, _NENS)
    acomb = jnp.concatenate([a1p, wall], axis=1)
    a2p = _build_a2p(W2)
    a3p = _build_a3p(W3)
    b2v = jnp.pad(jnp.repeat(b2, 25), (0, 78)).reshape(1, 128)
    b1r = b1.reshape(1, 1)
    b3r = b3.reshape(1, 4)
    blr = bl.reshape(1, _OUT_DIM)
    e_m = jnp.asarray(_E_EXPAND)
    s_m = jnp.asarray(_S_SUM)

    out = pl.pallas_call(
        _fwd_kernel,
        grid=(B // _BBLK,),
        in_specs=[
            pl.BlockSpec((_BBLK, _FLAT), lambda i: (i, 0)),
            pl.BlockSpec((_FLAT, _NCOMB), lambda i: (0, 0)),
            pl.BlockSpec((256, 512), lambda i: (0, 0)),
            pl.BlockSpec((128, 4), lambda i: (0, 0)),
            pl.BlockSpec((4, _OUT_DIM), lambda i: (0, 0)),
            pl.BlockSpec((1, 1), lambda i: (0, 0)),
            pl.BlockSpec((1, 128), lambda i: (0, 0)),
            pl.BlockSpec((1, 4), lambda i: (0, 0)),
            pl.BlockSpec((1, _OUT_DIM), lambda i: (0, 0)),
            pl.BlockSpec((_OUT_DIM, _NENS), lambda i: (0, 0)),
            pl.BlockSpec((_NENS, _NCLS), lambda i: (0, 0)),
            pl.BlockSpec((_OUT_DIM, _NCLS), lambda i: (0, 0)),
        ],
        out_specs=pl.BlockSpec((_BBLK, _NCLS), lambda i: (i, 0)),
        out_shape=jax.ShapeDtypeStruct((B, _NCLS), jnp.float32),
    )(xflat, acomb, a2p, a3p, Wl, b1r, b2v, b3r, blr, e_m, s_m, blearn)
    return out


# speedup vs baseline: 5.5233x; 1.1496x over previous
import numpy as np

import jax
import jax.numpy as jnp
from jax.experimental import pallas as pl

_B = 2048
_BBLK = 512
_FLAT = 3072
_NSEL = 1024
_NENS = 160
_NCOMB = _NSEL + _NENS
_K = 4
_OUT_DIM = 16
_NCLS = 10


def _sel(h_in, i_pad):
    d = np.arange(2)[:, None, None, None]
    y = np.arange(h_in)[None, :, None, None]
    i = np.arange(i_pad)[None, None, :, None]
    u = np.arange(5)[None, None, None, :]
    return (y == 2 * i + d + u).astype(np.float32)


_RY1 = _sel(32, 16)
_RY2 = _sel(16, 8)


def _build_a1p(W1):
    w = W1[0]
    r = jnp.asarray(_RY1)
    t1 = jnp.einsum("cuv,ayiu->cayiv", w, r)
    a1 = jnp.einsum("cayiv,bxjv->cyxabij", t1, r)
    return a1.reshape(_FLAT, _NSEL)


def _build_a2p(W2):
    w = W2[:, 0]
    r = jnp.asarray(_RY2)
    t1 = jnp.einsum("muv,ayiu->mayiv", w, r)
    a2 = jnp.einsum("mayiv,bxjv->yxabmij", t1, r)
    return a2.reshape(256, 512)


def _build_a3p(W3):
    a3 = jnp.pad(W3.transpose(1, 2, 3, 0), ((0, 0), (0, 3), (0, 3), (0, 0)))
    return a3.reshape(128, 4)


_E_EXPAND = np.kron(np.eye(_OUT_DIM, dtype=np.float32), np.ones((1, _NCLS), np.float32))
_S_SUM = np.kron(np.ones((_OUT_DIM, 1), np.float32), np.eye(_NCLS, dtype=np.float32))


def _fwd_kernel(x_ref, a1p_ref, wall_ref, a2p_ref, a3p_ref, wl_ref, b1_ref,
                b2v_ref, b3_ref, bl_ref, e_ref, s_ref, blearn_ref, out_ref):
    xb = x_ref[...]
    y1 = jnp.dot(xb, a1p_ref[...], preferred_element_type=jnp.float32)
    g = jnp.dot(xb, wall_ref[...], preferred_element_type=jnp.float32)
    p1 = jnp.maximum(jnp.maximum(y1[:, 0:256], y1[:, 256:512]),
                     jnp.maximum(y1[:, 512:768], y1[:, 768:1024]))
    p1 = jax.nn.relu(p1 + b1_ref[0, 0])
    y2 = jnp.dot(p1, a2p_ref[...], preferred_element_type=jnp.float32)
    p2 = jnp.maximum(jnp.maximum(y2[:, 0:128], y2[:, 128:256]),
                     jnp.maximum(y2[:, 256:384], y2[:, 384:512]))
    p2 = jax.nn.relu(p2 + b2v_ref[...])
    h4 = jax.nn.relu(jnp.dot(p2, a3p_ref[...], preferred_element_type=jnp.float32)
                     + b3_ref[...])
    s = jnp.dot(h4, wl_ref[...], preferred_element_type=jnp.float32) + bl_ref[...]
    sp = s[:, :, None]
    sl = s[:, None, :]
    ip = jax.lax.broadcasted_iota(jnp.int32, (s.shape[0], 16, 16), 1)
    il = jax.lax.broadcasted_iota(jnp.int32, (s.shape[0], 16, 16), 2)
    pred = (sp > sl) | ((sp == sl) & (ip < il))
    cnt = pred.astype(jnp.float32).sum(axis=1)
    mask = (cnt < (_K - 0.5)).astype(jnp.float32)
    maske = jnp.dot(mask, e_ref[...], preferred_element_type=jnp.float32)
    out = jnp.dot(g * maske, s_ref[...], preferred_element_type=jnp.float32)
    out = out + jnp.dot(mask, blearn_ref[...], preferred_element_type=jnp.float32)
    out_ref[...] = out


def kernel(x, W1, b1, W2, b2, W3, b3, Wl, bl, Wlearn, blearn):
    B = x.shape[0]
    xflat = x.reshape(B, _FLAT)
    a1p = _build_a1p(W1)
    wall = Wlearn.transpose(1, 0, 2).reshape(_FLAT, _NENS)
    a2p = _build_a2p(W2)
    a3p = _build_a3p(W3)
    b2v = jnp.repeat(b2, 64).reshape(1, 128)
    b1r = b1.reshape(1, 1)
    b3r = b3.reshape(1, 4)
    blr = bl.reshape(1, _OUT_DIM)
    e_m = jnp.asarray(_E_EXPAND)
    s_m = jnp.asarray(_S_SUM)

    out = pl.pallas_call(
        _fwd_kernel,
        grid=(B // _BBLK,),
        in_specs=[
            pl.BlockSpec((_BBLK, _FLAT), lambda i: (i, 0)),
            pl.BlockSpec((_FLAT, _NSEL), lambda i: (0, 0)),
            pl.BlockSpec((_FLAT, _NENS), lambda i: (0, 0)),
            pl.BlockSpec((256, 512), lambda i: (0, 0)),
            pl.BlockSpec((128, 4), lambda i: (0, 0)),
            pl.BlockSpec((4, _OUT_DIM), lambda i: (0, 0)),
            pl.BlockSpec((1, 1), lambda i: (0, 0)),
            pl.BlockSpec((1, 128), lambda i: (0, 0)),
            pl.BlockSpec((1, 4), lambda i: (0, 0)),
            pl.BlockSpec((1, _OUT_DIM), lambda i: (0, 0)),
            pl.BlockSpec((_OUT_DIM, _NENS), lambda i: (0, 0)),
            pl.BlockSpec((_NENS, _NCLS), lambda i: (0, 0)),
            pl.BlockSpec((_OUT_DIM, _NCLS), lambda i: (0, 0)),
        ],
        out_specs=pl.BlockSpec((_BBLK, _NCLS), lambda i: (i, 0)),
        out_shape=jax.ShapeDtypeStruct((B, _NCLS), jnp.float32),
    )(xflat, a1p, wall, a2p, a3p, Wl, b1r, b2v, b3r, blr, e_m, s_m, blearn)
    return out
